# baseline (speedup 1.0000x reference)
import functools

import jax
import jax.numpy as jnp
from jax import lax
from jax.experimental import pallas as pl
from jax.experimental.pallas import tpu as pltpu
from jax.experimental.pallas import tpu_sc as plsc

B = 4096
GT, GN, GD = 13, 100000, 64
CT, CN, CD = 13, 1000000, 32
DF = 13
OD = 512

NW = 32
ROWS = B * GT
R_PER_W = ROWS // NW

_sc_mesh = plsc.VectorSubcoreMesh(core_axis_name="c", subcore_axis_name="s")


G = 16
NGRP = R_PER_W // G


@functools.partial(
    pl.kernel,
    mesh=_sc_mesh,
    out_type=jax.ShapeDtypeStruct((ROWS, GD), jnp.float32),
    scratch_types=[
        pltpu.VMEM((R_PER_W,), jnp.int32),
        pltpu.VMEM((R_PER_W,), jnp.int32),
        pltpu.VMEM((G * 8, GD), jnp.float32),
        pltpu.VMEM((G, GD), jnp.float32),
    ]
    + [pltpu.SemaphoreType.DMA] * (G + 3),
)
def _sc_gpu_gather(gtab, vb_h, sb_h, gout, vb_v, sb_v, blk, rows, *sems):
    osem = sems[G + 2]
    wid = lax.axis_index("s") * 2 + lax.axis_index("c")
    base = wid * R_PER_W

    h0 = pltpu.async_copy(vb_h.at[wid], vb_v, sems[G])
    h1 = pltpu.async_copy(sb_h.at[wid], sb_v, sems[G + 1])
    h0.wait(); h1.wait()

    def group(j, carry):
        bv = vb_v[pl.ds(j * G, G)]
        sv = sb_v[pl.ds(j * G, G)]
        for u in range(G):
            vb = pl.multiple_of(bv[u], 8)
            pltpu.async_copy(gtab.at[pl.ds(vb, 8)],
                             blk.at[pl.ds(u * 8, 8)], sems[u])
        for u in range(G):
            pltpu.make_async_copy(gtab.at[pl.ds(0, 8)],
                                  blk.at[pl.ds(u * 8, 8)], sems[u]).wait()
            sub = sv[u]
            for k in range(GD // 16):
                rows[u, pl.ds(k * 16, 16)] = blk[u * 8 + sub, pl.ds(k * 16, 16)]
        pltpu.async_copy(rows, gout.at[pl.ds(base + j * G, G)], osem).wait()
        return carry

    lax.fori_loop(0, NGRP, group, 0)


@functools.partial(
    pl.kernel,
    mesh=_sc_mesh,
    out_type=jax.ShapeDtypeStruct((ROWS, CD), jnp.float32),
    scratch_types=[
        pltpu.VMEM((R_PER_W,), jnp.int32),
        pltpu.VMEM((R_PER_W,), jnp.int32),
        pltpu.VMEM((R_PER_W,), jnp.int32),
        pltpu.VMEM((G * CD, 128), jnp.float32),
        pltpu.VMEM((G, CD), jnp.float32),
    ]
    + [pltpu.SemaphoreType.DMA] * (G + 4),
)
def _sc_cpu_gather(tabT, t_h, vb_h, vl_h, out, t_v, vb_v, vl_v, blk, rows, *sems):
    osem = sems[G + 3]
    wid = lax.axis_index("s") * 2 + lax.axis_index("c")
    base = wid * R_PER_W

    h0 = pltpu.async_copy(t_h.at[wid], t_v, sems[G])
    h1 = pltpu.async_copy(vb_h.at[wid], vb_v, sems[G + 1])
    h2 = pltpu.async_copy(vl_h.at[wid], vl_v, sems[G + 2])
    h0.wait(); h1.wait(); h2.wait()

    lane = lax.iota(jnp.int32, 16)

    def group(j, carry):
        tv = t_v[pl.ds(j * G, G)]
        bv = vb_v[pl.ds(j * G, G)]
        lv = vl_v[pl.ds(j * G, G)]
        for u in range(G):
            t = tv[u]
            vb = pl.multiple_of(bv[u], 128)
            pltpu.async_copy(tabT.at[t, :, pl.ds(vb, 128)],
                             blk.at[pl.ds(u * CD, CD)], sems[u])
        for u in range(G):
            pltpu.make_async_copy(tabT.at[0, :, pl.ds(0, 128)],
                                  blk.at[pl.ds(u * CD, CD)], sems[u]).wait()
            g16 = (lv[u] // 16) * 16
            lam = lv[lane * 0 + u] & 15
            acc0 = (lane * 0).astype(jnp.float32)
            acc1 = (lane * 0).astype(jnp.float32)
            for r in range(CD):
                x = blk[u * CD + r, pl.ds(g16, 16)]
                s = x[lam]
                if r < 16:
                    acc0 = jnp.where(lane == r, s, acc0)
                else:
                    acc1 = jnp.where(lane == (r - 16), s, acc1)
            rows[u, pl.ds(0, 16)] = acc0
            rows[u, pl.ds(16, 16)] = acc1
        pltpu.async_copy(rows, out.at[pl.ds(base + j * G, G)], osem).wait()
        return carry

    lax.fori_loop(0, NGRP, group, 0)


def _mm(x, w):
    return lax.dot_general(x, w, (((1,), (1,)), ((), ())),
                           preferred_element_type=jnp.float32)


def _tc_body(ge, ce, df, dw1, db1, dw2, db2, w1g, w1c, w1d, ob1,
             ow2, ob2, ow3, ob3, ow4, ob4, ow5, ob5, out):
    h = jnp.maximum(_mm(df[...], dw1[...]) + db1[...], 0.0)
    de = _mm(h, dw2[...]) + db2[...]
    o = _mm(ge[...], w1g[...]) + _mm(ce[...], w1c[...]) + _mm(de, w1d[...]) + ob1[...]
    o = jnp.maximum(o, 0.0)
    o = jnp.maximum(_mm(o, ow2[...]) + ob2[...], 0.0)
    o = jnp.maximum(_mm(o, ow3[...]) + ob3[...], 0.0)
    o = jnp.maximum(_mm(o, ow4[...]) + ob4[...], 0.0)
    out[...] = _mm(o, ow5[...]) + ob5[...]


BLK = 512


def _full(a):
    return pl.BlockSpec(a.shape, lambda i: (0,) * a.ndim)


def _tc_forward(ge, ce, df, dw1, db1, dw2, db2, w1g, w1c, w1d, ob1,
                ow2, ob2, ow3, ob3, ow4, ob4, ow5, ob5):
    grid = (B // BLK,)
    in_specs = [
        pl.BlockSpec((BLK, GT * GD), lambda i: (i, 0)),
        pl.BlockSpec((BLK, CT * CD), lambda i: (i, 0)),
        pl.BlockSpec((BLK, DF), lambda i: (i, 0)),
    ] + [_full(a) for a in (dw1, db1, dw2, db2, w1g, w1c, w1d, ob1,
                            ow2, ob2, ow3, ob3, ow4, ob4, ow5, ob5)]
    return pl.pallas_call(
        _tc_body,
        grid=grid,
        in_specs=in_specs,
        out_specs=pl.BlockSpec((BLK, 128), lambda i: (i, 0)),
        out_shape=jax.ShapeDtypeStruct((B, 128), jnp.float32),
    )(ge, ce, df, dw1, db1, dw2, db2, w1g, w1c, w1d, ob1,
      ow2, ob2, ow3, ob3, ow4, ob4, ow5, ob5)


def kernel(dense_features, gpu_sharded_sparse_features, cpu_sharded_sparse_features,
           gpu_tables, cpu_tables, dw1, db1, dw2, db2,
           ow1, ob1, ow2, ob2, ow3, ob3, ow4, ob4, ow5, ob5):
    gidx = gpu_sharded_sparse_features.astype(jnp.int32)
    cidx = cpu_sharded_sparse_features.astype(jnp.int32)
    gflat = (gidx + jnp.arange(GT, dtype=jnp.int32)[None, :] * GN)
    gvb = ((gflat // 8) * 8).reshape(NW, R_PER_W)
    gsb = (gflat % 8).reshape(NW, R_PER_W)
    gout = _sc_gpu_gather(gpu_tables.reshape(GT * GN, GD), gvb, gsb)
    ge = gout.reshape(B, GT * GD)

    ctabT = cpu_tables.transpose(0, 2, 1)
    t_arr = jnp.broadcast_to(jnp.arange(CT, dtype=jnp.int32)[None, :], (B, CT))
    vb_arr = (cidx // 128) * 128
    vl_arr = cidx - vb_arr
    cout = _sc_cpu_gather(ctabT,
                          t_arr.reshape(NW, R_PER_W),
                          vb_arr.reshape(NW, R_PER_W),
                          vl_arr.reshape(NW, R_PER_W))
    ce = cout.reshape(B, CT * CD)

    w1g = ow1[:, : GT * GD]
    w1c = ow1[:, GT * GD: GT * GD + CT * CD]
    w1d = ow1[:, GT * GD + CT * CD:]

    ow5p = jnp.pad(ow5, ((0, 127), (0, 0)))
    ob5p = jnp.pad(ob5.reshape(1, 1), ((0, 0), (0, 127)))
    out = _tc_forward(ge, ce, dense_features,
                      dw1, db1.reshape(1, OD), dw2, db2.reshape(1, GD),
                      w1g, w1c, w1d, ob1.reshape(1, OD),
                      ow2, ob2.reshape(1, OD), ow3, ob3.reshape(1, OD),
                      ow4, ob4.reshape(1, OD), ow5p, ob5p)
    return out[:, :1]


# speedup vs baseline: 1.1324x; 1.1324x over previous
import functools

import jax
import jax.numpy as jnp
from jax import lax
from jax.experimental import pallas as pl
from jax.experimental.pallas import tpu as pltpu
from jax.experimental.pallas import tpu_sc as plsc

B = 4096
GT, GN, GD = 13, 100000, 64
CT, CN, CD = 13, 1000000, 32
DF = 13
OD = 512

NW = 32
ROWS = B * GT
R_PER_W = ROWS // NW

_sc_mesh = plsc.VectorSubcoreMesh(core_axis_name="c", subcore_axis_name="s")


G = 16
NGRP = R_PER_W // G


@functools.partial(
    pl.kernel,
    mesh=_sc_mesh,
    out_type=[jax.ShapeDtypeStruct((ROWS, GD), jnp.float32),
              jax.ShapeDtypeStruct((ROWS, CD), jnp.float32)],
    scratch_types=[
        pltpu.VMEM((R_PER_W,), jnp.int32),
        pltpu.VMEM((R_PER_W,), jnp.int32),
        pltpu.VMEM((R_PER_W,), jnp.int32),
        pltpu.VMEM((R_PER_W,), jnp.int32),
        pltpu.VMEM((R_PER_W,), jnp.int32),
        pltpu.VMEM((G * 8, GD), jnp.float32),
        pltpu.VMEM((G * CD, 128), jnp.float32),
        pltpu.VMEM((G, GD), jnp.float32),
        pltpu.VMEM((G, CD), jnp.float32),
    ]
    + [pltpu.SemaphoreType.DMA] * 4,
)
def _sc_gather(gtab, gvb_h, gsb_h, ctabT, ct_h, cvb_h, cvl_h,
               gout, cout, gvb_v, gsb_v, ct_v, cvb_v, cvl_v,
               gblk, cblk, grows, crows, *sems):
    gsem, csem, lsem, osem = sems
    wid = lax.axis_index("s") * 2 + lax.axis_index("c")
    base = wid * R_PER_W

    hs = [pltpu.async_copy(h.at[wid], v, lsem) for h, v in
          ((gvb_h, gvb_v), (gsb_h, gsb_v), (ct_h, ct_v),
           (cvb_h, cvb_v), (cvl_h, cvl_v))]
    for h in hs:
        h.wait()

    lane = lax.iota(jnp.int32, 16)

    def group(j, carry):
        gbv = gvb_v[pl.ds(j * G, G)]
        gsv = gsb_v[pl.ds(j * G, G)]
        tv = ct_v[pl.ds(j * G, G)]
        bv = cvb_v[pl.ds(j * G, G)]
        lv = cvl_v[pl.ds(j * G, G)]
        ch = []
        gh = []
        for u in range(G):
            t = tv[u]
            vb = pl.multiple_of(bv[u], 128)
            ch.append(pltpu.async_copy(ctabT.at[t, :, pl.ds(vb, 128)],
                                       cblk.at[pl.ds(u * CD, CD)], csem))
        for u in range(G):
            gvb = pl.multiple_of(gbv[u], 8)
            gh.append(pltpu.async_copy(gtab.at[pl.ds(gvb, 8)],
                                       gblk.at[pl.ds(u * 8, 8)], gsem))
        for h in gh:
            h.wait()
        for u in range(G):
            sub = gsv[u]
            for k in range(GD // 16):
                grows[u, pl.ds(k * 16, 16)] = gblk[u * 8 + sub, pl.ds(k * 16, 16)]
        hg = pltpu.async_copy(grows, gout.at[pl.ds(base + j * G, G)], osem)
        for h in ch:
            h.wait()
        for u in range(G):
            g16 = (lv[u] // 16) * 16
            lam = lv[lane * 0 + u] & 15
            acc0 = (lane * 0).astype(jnp.float32)
            acc1 = (lane * 0).astype(jnp.float32)
            for r in range(CD):
                x = cblk[u * CD + r, pl.ds(g16, 16)]
                s = x[lam]
                if r < 16:
                    acc0 = jnp.where(lane == r, s, acc0)
                else:
                    acc1 = jnp.where(lane == (r - 16), s, acc1)
            crows[u, pl.ds(0, 16)] = acc0
            crows[u, pl.ds(16, 16)] = acc1
        hg.wait()
        pltpu.async_copy(crows, cout.at[pl.ds(base + j * G, G)], osem).wait()
        return carry

    lax.fori_loop(0, NGRP, group, 0)


def _mm(x, w):
    return lax.dot_general(x, w, (((1,), (1,)), ((), ())),
                           preferred_element_type=jnp.float32)


def _tc_body(ge, ce, df, dw1, db1, dw2, db2, w1g, w1c, w1d, ob1,
             ow2, ob2, ow3, ob3, ow4, ob4, ow5, ob5, out):
    h = jnp.maximum(_mm(df[...], dw1[...]) + db1[...], 0.0)
    de = _mm(h, dw2[...]) + db2[...]
    o = _mm(ge[...], w1g[...]) + _mm(ce[...], w1c[...]) + _mm(de, w1d[...]) + ob1[...]
    o = jnp.maximum(o, 0.0)
    o = jnp.maximum(_mm(o, ow2[...]) + ob2[...], 0.0)
    o = jnp.maximum(_mm(o, ow3[...]) + ob3[...], 0.0)
    o = jnp.maximum(_mm(o, ow4[...]) + ob4[...], 0.0)
    out[...] = _mm(o, ow5[...]) + ob5[...]


BLK = 512


def _full(a):
    return pl.BlockSpec(a.shape, lambda i: (0,) * a.ndim)


def _tc_forward(ge, ce, df, dw1, db1, dw2, db2, w1g, w1c, w1d, ob1,
                ow2, ob2, ow3, ob3, ow4, ob4, ow5, ob5):
    grid = (B // BLK,)
    in_specs = [
        pl.BlockSpec((BLK, GT * GD), lambda i: (i, 0)),
        pl.BlockSpec((BLK, CT * CD), lambda i: (i, 0)),
        pl.BlockSpec((BLK, DF), lambda i: (i, 0)),
    ] + [_full(a) for a in (dw1, db1, dw2, db2, w1g, w1c, w1d, ob1,
                            ow2, ob2, ow3, ob3, ow4, ob4, ow5, ob5)]
    return pl.pallas_call(
        _tc_body,
        grid=grid,
        in_specs=in_specs,
        out_specs=pl.BlockSpec((BLK, 128), lambda i: (i, 0)),
        out_shape=jax.ShapeDtypeStruct((B, 128), jnp.float32),
    )(ge, ce, df, dw1, db1, dw2, db2, w1g, w1c, w1d, ob1,
      ow2, ob2, ow3, ob3, ow4, ob4, ow5, ob5)


def kernel(dense_features, gpu_sharded_sparse_features, cpu_sharded_sparse_features,
           gpu_tables, cpu_tables, dw1, db1, dw2, db2,
           ow1, ob1, ow2, ob2, ow3, ob3, ow4, ob4, ow5, ob5):
    gidx = gpu_sharded_sparse_features.astype(jnp.int32)
    cidx = cpu_sharded_sparse_features.astype(jnp.int32)
    gflat = (gidx + jnp.arange(GT, dtype=jnp.int32)[None, :] * GN)
    gvb = ((gflat // 8) * 8).reshape(NW, R_PER_W)
    gsb = (gflat % 8).reshape(NW, R_PER_W)

    ctabT = cpu_tables.transpose(0, 2, 1)
    t_arr = jnp.broadcast_to(jnp.arange(CT, dtype=jnp.int32)[None, :], (B, CT))
    vb_arr = (cidx // 128) * 128
    vl_arr = cidx - vb_arr

    gout, cout = _sc_gather(gpu_tables.reshape(GT * GN, GD), gvb, gsb,
                            ctabT,
                            t_arr.reshape(NW, R_PER_W),
                            vb_arr.reshape(NW, R_PER_W),
                            vl_arr.reshape(NW, R_PER_W))
    ge = gout.reshape(B, GT * GD)
    ce = cout.reshape(B, CT * CD)

    w1g = ow1[:, : GT * GD]
    w1c = ow1[:, GT * GD: GT * GD + CT * CD]
    w1d = ow1[:, GT * GD + CT * CD:]

    ow5p = jnp.pad(ow5, ((0, 127), (0, 0)))
    ob5p = jnp.pad(ob5.reshape(1, 1), ((0, 0), (0, 127)))
    out = _tc_forward(ge, ce, dense_features,
                      dw1, db1.reshape(1, OD), dw2, db2.reshape(1, GD),
                      w1g, w1c, w1d, ob1.reshape(1, OD),
                      ow2, ob2.reshape(1, OD), ow3, ob3.reshape(1, OD),
                      ow4, ob4.reshape(1, OD), ow5p, ob5p)
    return out[:, :1]


# speedup vs baseline: 1.2571x; 1.1101x over previous
import functools

import jax
import jax.numpy as jnp
from jax import lax
from jax.experimental import pallas as pl
from jax.experimental.pallas import tpu as pltpu
from jax.experimental.pallas import tpu_sc as plsc

B = 4096
GT, GN, GD = 13, 100000, 64
CT, CN, CD = 13, 1000000, 32
DF = 13
OD = 512

NW = 32
ROWS = B * GT
R_PER_W = ROWS // NW

_sc_mesh = plsc.VectorSubcoreMesh(core_axis_name="c", subcore_axis_name="s")


G = 8
NGRP = R_PER_W // G


@functools.partial(
    pl.kernel,
    mesh=_sc_mesh,
    out_type=[jax.ShapeDtypeStruct((ROWS, GD), jnp.float32),
              jax.ShapeDtypeStruct((ROWS, CD), jnp.float32)],
    scratch_types=[
        pltpu.VMEM((R_PER_W,), jnp.int32),
        pltpu.VMEM((R_PER_W,), jnp.int32),
        pltpu.VMEM((R_PER_W,), jnp.int32),
        pltpu.VMEM((R_PER_W,), jnp.int32),
        pltpu.VMEM((R_PER_W,), jnp.int32),
        pltpu.VMEM((2, G * 8, GD), jnp.float32),
        pltpu.VMEM((2, G * CD, 128), jnp.float32),
        pltpu.VMEM((G, GD), jnp.float32),
        pltpu.VMEM((G, CD), jnp.float32),
    ]
    + [pltpu.SemaphoreType.DMA] * 5,
)
def _sc_gather(gtab, gvb_h, gsb_h, ctabT, ct_h, cvb_h, cvl_h,
               gout, cout, gvb_v, gsb_v, ct_v, cvb_v, cvl_v,
               gblk, cblk, grows, crows, *sems):
    gsem0, csem0, gsem1, csem1, osem = sems
    wid = lax.axis_index("s") * 2 + lax.axis_index("c")
    base = wid * R_PER_W

    hs = [pltpu.async_copy(h.at[wid], v, osem) for h, v in
          ((gvb_h, gvb_v), (gsb_h, gsb_v), (ct_h, ct_v),
           (cvb_h, cvb_v), (cvl_h, cvl_v))]
    for h in hs:
        h.wait()

    lane = lax.iota(jnp.int32, 16)

    def fire(j, bank, gsem, csem):
        tv = ct_v[pl.ds(j * G, G)]
        bv = cvb_v[pl.ds(j * G, G)]
        gbv = gvb_v[pl.ds(j * G, G)]
        for u in range(G):
            t = tv[u]
            vb = pl.multiple_of(bv[u], 128)
            pltpu.async_copy(ctabT.at[t, :, pl.ds(vb, 128)],
                             cblk.at[bank].at[pl.ds(u * CD, CD)], csem)
        for u in range(G):
            gvb = pl.multiple_of(gbv[u], 8)
            pltpu.async_copy(gtab.at[pl.ds(gvb, 8)],
                             gblk.at[bank].at[pl.ds(u * 8, 8)], gsem)

    def drain(j, bank, gsem, csem):
        gsv = gsb_v[pl.ds(j * G, G)]
        lv = cvl_v[pl.ds(j * G, G)]
        for u in range(G):
            pltpu.make_async_copy(gtab.at[pl.ds(0, 8)],
                                  gblk.at[bank].at[pl.ds(u * 8, 8)], gsem).wait()
        for u in range(G):
            sub = gsv[u]
            for k in range(GD // 16):
                grows[u, pl.ds(k * 16, 16)] = gblk[bank, u * 8 + sub, pl.ds(k * 16, 16)]
        hg = pltpu.async_copy(grows, gout.at[pl.ds(base + j * G, G)], osem)
        for u in range(G):
            pltpu.make_async_copy(ctabT.at[0, :, pl.ds(0, 128)],
                                  cblk.at[bank].at[pl.ds(u * CD, CD)], csem).wait()
        for u in range(G):
            g16 = (lv[u] // 16) * 16
            lam = lv[lane * 0 + u] & 15
            acc0 = (lane * 0).astype(jnp.float32)
            acc1 = (lane * 0).astype(jnp.float32)
            for r in range(CD):
                x = cblk[bank, u * CD + r, pl.ds(g16, 16)]
                s = x[lam]
                if r < 16:
                    acc0 = jnp.where(lane == r, s, acc0)
                else:
                    acc1 = jnp.where(lane == (r - 16), s, acc1)
            crows[u, pl.ds(0, 16)] = acc0
            crows[u, pl.ds(16, 16)] = acc1
        hg.wait()
        pltpu.async_copy(crows, cout.at[pl.ds(base + j * G, G)], osem).wait()

    fire(0, 0, gsem0, csem0)

    def pair(k, carry):
        fire(2 * k + 1, 1, gsem1, csem1)
        drain(2 * k, 0, gsem0, csem0)

        @pl.when(2 * k + 2 < NGRP)
        def _():
            fire(2 * k + 2, 0, gsem0, csem0)

        drain(2 * k + 1, 1, gsem1, csem1)
        return carry

    lax.fori_loop(0, NGRP // 2, pair, 0)


def _mm(x, w):
    return lax.dot_general(x, w, (((1,), (1,)), ((), ())),
                           preferred_element_type=jnp.float32)


def _tc_body(ge, ce, df, dw1, db1, dw2, db2, w1g, w1c, w1d, ob1,
             ow2, ob2, ow3, ob3, ow4, ob4, ow5, ob5, out):
    h = jnp.maximum(_mm(df[...], dw1[...]) + db1[...], 0.0)
    de = _mm(h, dw2[...]) + db2[...]
    o = _mm(ge[...], w1g[...]) + _mm(ce[...], w1c[...]) + _mm(de, w1d[...]) + ob1[...]
    o = jnp.maximum(o, 0.0)
    o = jnp.maximum(_mm(o, ow2[...]) + ob2[...], 0.0)
    o = jnp.maximum(_mm(o, ow3[...]) + ob3[...], 0.0)
    o = jnp.maximum(_mm(o, ow4[...]) + ob4[...], 0.0)
    out[...] = _mm(o, ow5[...]) + ob5[...]


BLK = 512


def _full(a):
    return pl.BlockSpec(a.shape, lambda i: (0,) * a.ndim)


def _tc_forward(ge, ce, df, dw1, db1, dw2, db2, w1g, w1c, w1d, ob1,
                ow2, ob2, ow3, ob3, ow4, ob4, ow5, ob5):
    grid = (B // BLK,)
    in_specs = [
        pl.BlockSpec((BLK, GT * GD), lambda i: (i, 0)),
        pl.BlockSpec((BLK, CT * CD), lambda i: (i, 0)),
        pl.BlockSpec((BLK, DF), lambda i: (i, 0)),
    ] + [_full(a) for a in (dw1, db1, dw2, db2, w1g, w1c, w1d, ob1,
                            ow2, ob2, ow3, ob3, ow4, ob4, ow5, ob5)]
    return pl.pallas_call(
        _tc_body,
        grid=grid,
        in_specs=in_specs,
        out_specs=pl.BlockSpec((BLK, 128), lambda i: (i, 0)),
        out_shape=jax.ShapeDtypeStruct((B, 128), jnp.float32),
    )(ge, ce, df, dw1, db1, dw2, db2, w1g, w1c, w1d, ob1,
      ow2, ob2, ow3, ob3, ow4, ob4, ow5, ob5)


def kernel(dense_features, gpu_sharded_sparse_features, cpu_sharded_sparse_features,
           gpu_tables, cpu_tables, dw1, db1, dw2, db2,
           ow1, ob1, ow2, ob2, ow3, ob3, ow4, ob4, ow5, ob5):
    gidx = gpu_sharded_sparse_features.astype(jnp.int32)
    cidx = cpu_sharded_sparse_features.astype(jnp.int32)
    gflat = (gidx + jnp.arange(GT, dtype=jnp.int32)[None, :] * GN)
    gvb = ((gflat // 8) * 8).reshape(NW, R_PER_W)
    gsb = (gflat % 8).reshape(NW, R_PER_W)

    ctabT = cpu_tables.transpose(0, 2, 1)
    t_arr = jnp.broadcast_to(jnp.arange(CT, dtype=jnp.int32)[None, :], (B, CT))
    vb_arr = (cidx // 128) * 128
    vl_arr = cidx - vb_arr

    gout, cout = _sc_gather(gpu_tables.reshape(GT * GN, GD), gvb, gsb,
                            ctabT,
                            t_arr.reshape(NW, R_PER_W),
                            vb_arr.reshape(NW, R_PER_W),
                            vl_arr.reshape(NW, R_PER_W))
    ge = gout.reshape(B, GT * GD)
    ce = cout.reshape(B, CT * CD)

    w1g = ow1[:, : GT * GD]
    w1c = ow1[:, GT * GD: GT * GD + CT * CD]
    w1d = ow1[:, GT * GD + CT * CD:]

    ow5p = jnp.pad(ow5, ((0, 127), (0, 0)))
    ob5p = jnp.pad(ob5.reshape(1, 1), ((0, 0), (0, 127)))
    out = _tc_forward(ge, ce, dense_features,
                      dw1, db1.reshape(1, OD), dw2, db2.reshape(1, GD),
                      w1g, w1c, w1d, ob1.reshape(1, OD),
                      ow2, ob2.reshape(1, OD), ow3, ob3.reshape(1, OD),
                      ow4, ob4.reshape(1, OD), ow5p, ob5p)
    return out[:, :1]
